# baseline (speedup 1.0000x reference)
import functools

import jax
import jax.numpy as jnp
from jax import lax
from jax.experimental import pallas as pl
from jax.experimental.pallas import tpu as pltpu
from jax.experimental.pallas import tpu_sc as plsc

_N, _D = 8192, 768
_PATHS = 8
_NC, _NS, _L = 2, 16, 16
_NW = _NC * _NS
_RPW = _N // _NW
_ZR = 32
_NPAD = _RPW // _ZR
_NCH = 8
_CH = _N // _NCH

_mesh = plsc.VectorSubcoreMesh(core_axis_name="c", subcore_axis_name="s")


@functools.partial(
    pl.kernel,
    mesh=_mesh,
    out_type=jax.ShapeDtypeStruct((_N, _D), jnp.float32),
    scratch_types=[
        pltpu.VMEM((_ZR, _D), jnp.float32),
        pltpu.SemaphoreType.DMA,
    ],
)
def _sc_pad(pad_hbm, zbuf, pad_sem):
    wid = lax.axis_index("s") * _NC + lax.axis_index("c")
    base = wid * _RPW

    zvec = jnp.zeros((_L,), jnp.float32)

    def _zrow(i, carry):
        def _zcol(j, c):
            zbuf[i, pl.ds(j * _L, _L)] = zvec
            return c
        return lax.fori_loop(0, _D // _L, _zcol, carry)

    lax.fori_loop(0, _ZR, _zrow, 0)

    pads = [
        pltpu.async_copy(zbuf, pad_hbm.at[pl.ds(base + t * _ZR, _ZR), :],
                         pad_sem)
        for t in range(_NPAD)
    ]
    for p in pads:
        p.wait()


def _tc_hit_body(x_hbm, hit_hbm, sems):
    cps = [
        pltpu.make_async_copy(x_hbm.at[pl.ds(k * _CH, _CH), :],
                              hit_hbm.at[pl.ds(k * _CH, _CH), :],
                              sems.at[k])
        for k in range(_NCH)
    ]
    for c in cps:
        c.start()
    for c in cps:
        c.wait()


def kernel(inputs):
    n, d = inputs.shape
    pad = _sc_pad()
    hit = pl.pallas_call(
        _tc_hit_body,
        in_specs=[pl.BlockSpec(memory_space=pl.ANY)],
        out_specs=pl.BlockSpec(memory_space=pl.ANY),
        out_shape=jax.ShapeDtypeStruct((n, d), inputs.dtype),
        scratch_shapes=[pltpu.SemaphoreType.DMA((_NCH,))],
    )(inputs)
    return (hit,) + (pad,) * (_PATHS - 1)


# speedup vs baseline: 7.9423x; 7.9423x over previous
import functools

import jax
import jax.numpy as jnp
from jax import lax
from jax.experimental import pallas as pl
from jax.experimental.pallas import tpu as pltpu
from jax.experimental.pallas import tpu_sc as plsc

_N, _D = 8192, 768
_PATHS = 8
_NC, _NS, _L = 2, 16, 16
_NW = _NC * _NS
_RPW = _N // _NW
_ZR = 32
_NPAD = _RPW // _ZR
_NCH = 8
_CH = _N // _NCH

_mesh = plsc.VectorSubcoreMesh(core_axis_name="c", subcore_axis_name="s")


@functools.partial(
    pl.kernel,
    mesh=_mesh,
    out_type=jax.ShapeDtypeStruct((_N, _D), jnp.float32),
    scratch_types=[
        pltpu.VMEM((_ZR, _D), jnp.float32),
        pltpu.SemaphoreType.DMA,
    ],
)
def _sc_pad(pad_hbm, zbuf, pad_sem):
    wid = lax.axis_index("s") * _NC + lax.axis_index("c")
    base = wid * _RPW

    zvec = jnp.zeros((_L,), jnp.float32)

    def _zrow(i, carry):
        def _zcol(j, c):
            zbuf[i, pl.ds(j * _L, _L)] = zvec
            return c
        return lax.fori_loop(0, _D // _L, _zcol, carry)

    lax.fori_loop(0, _ZR, _zrow, 0)

    pads = [
        pltpu.async_copy(zbuf, pad_hbm.at[pl.ds(base + t * _ZR, _ZR), :],
                         pad_sem)
        for t in range(_NPAD)
    ]
    for p in pads:
        p.wait()


_BLOCK = 1024


def _copy_body(x_ref, hit_ref):
    hit_ref[...] = x_ref[...]


def kernel(inputs):
    n, d = inputs.shape
    hit = pl.pallas_call(
        _copy_body,
        grid=(n // _BLOCK,),
        in_specs=[pl.BlockSpec((_BLOCK, d), lambda i: (i, 0))],
        out_specs=pl.BlockSpec((_BLOCK, d), lambda i: (i, 0)),
        out_shape=jax.ShapeDtypeStruct((n, d), inputs.dtype),
    )(inputs)
    pad = _sc_pad()
    return (hit,) + (pad,) * (_PATHS - 1)
